# Initial kernel scaffold; baseline (speedup 1.0000x reference)
import jax
import jax.numpy as jnp
from jax.experimental import pallas as pl


def kernel(feats, points, Wv1, bv1, Wv2, bv2, W1a, W1b, W1s, W2a, W2b, W2s, W3a, W3b, W3s, pos_length, anc_length):
    raise NotImplementedError("write your pallas kernel here")


# speedup vs baseline: 13.8658x; 13.8658x over previous
import functools

import jax
import jax.numpy as jnp
from jax import lax
from jax.experimental import pallas as pl
from jax.experimental.pallas import tpu as pltpu

NPAD = 5120
B = 256
HI = lax.Precision.HIGHEST
NB = NPAD // B
K = 16
GROUPS = 32
R2_NMS = 0.1 * 0.1
R2_SUB = (0.0625 * 8.0) ** 2
R2_NB = (0.0625 * 16.0) ** 2


def _a2col(q):
    xq, yq, zq = q[:, 0:1], q[:, 1:2], q[:, 2:3]
    return (xq * xq + zq * zq) + yq * yq


def _b2row(st):
    xs, ys, zs = st[0:1, :], st[1:2, :], st[2:3, :]
    return (xs * xs + zs * zs) + ys * ys


def _vote_body(f_ref, p_ref, w1_ref, b1_ref, w2_ref, b2_ref, out_ref):
    x = f_ref[...]
    h = jnp.maximum(
        jnp.dot(x, w1_ref[...], preferred_element_type=jnp.float32) + b1_ref[...], 0.0)
    off = jnp.dot(h, w2_ref[...], preferred_element_type=jnp.float32) + b2_ref[...]
    score = jnp.sqrt(jnp.sum(x * x, axis=1, keepdims=True))
    col = lax.broadcasted_iota(jnp.int32, (1, 128), 1)
    out_ref[...] = off + p_ref[...] + jnp.where(col == 3, score, 0.0)


def _nms_body(sp_ref, spt_ref, seg_ref, segt_ref, sptb_ref, segtb_ref,
              keep_ref, len_ref, kv_ref, cnt_ref):
    b = pl.program_id(0)
    nb = pl.num_programs(0)

    @pl.when(b == 0)
    def _init():
        kv_ref[...] = jnp.zeros_like(kv_ref)
        cnt_ref[0] = 0.0
        cnt_ref[1] = 0.0

    q = sp_ref[...]
    st = spt_ref[...]
    a2 = _a2col(q)
    b2 = _b2row(st)
    d2 = jnp.maximum(
        a2 + b2 - 2.0 * jnp.dot(q, st, preferred_element_type=jnp.float32), 0.0)
    segq = seg_ref[...]
    adj = jnp.where((d2 < R2_NMS) & (segq == segt_ref[...]), 1.0, 0.0)
    sprior = jnp.dot(adj, kv_ref[...], preferred_element_type=jnp.float32)
    notprior = jnp.where(sprior > 0.0, 0.0, 1.0)

    stb = sptb_ref[...]
    b2in = _b2row(stb)
    d2in = jnp.maximum(
        a2 + b2in - 2.0 * jnp.dot(q, stb, preferred_element_type=jnp.float32), 0.0)
    rr = lax.broadcasted_iota(jnp.int32, (B, B), 0)
    cc = lax.broadcasted_iota(jnp.int32, (B, B), 1)
    adj_low = jnp.where(
        (d2in < R2_NMS) & (segq == segtb_ref[...]) & (cc < rr), 1.0, 0.0)

    def _cond(c):
        return c[1]

    def _body(c):
        kb, _ = c
        s = jnp.dot(adj_low, kb, preferred_element_type=jnp.float32)
        knew = notprior * jnp.where(s > 0.0, 0.0, 1.0)
        return knew, jnp.sum(jnp.abs(knew - kb)) > 0.0

    kb, _ = lax.while_loop(_cond, _body, (notprior, jnp.asarray(True)))

    kv_ref[pl.ds(b * B, B), :] = kb
    keep_ref[...] = kb
    cnt_ref[0] = cnt_ref[0] + jnp.sum(kb * jnp.where(segq == 0.0, 1.0, 0.0))
    cnt_ref[1] = cnt_ref[1] + jnp.sum(kb * jnp.where(segq == 1.0, 1.0, 0.0))

    @pl.when(b == nb - 1)
    def _fin():
        colv = lax.broadcasted_iota(jnp.int32, (8, 128), 1)
        len_ref[...] = jnp.where(
            colv == 0, cnt_ref[0], jnp.where(colv == 1, cnt_ref[1], 0.0))


def _knn_body(q_ref, st_ref, segq_ref, segt_ref, sv_ref, out_ref):
    q = q_ref[...]
    st = st_ref[...]
    a2 = _a2col(q)
    b2 = _b2row(st)
    d2 = jnp.maximum(
        a2 + b2 - 2.0 * jnp.dot(q, st, preferred_element_type=jnp.float32), 0.0)
    d2m = jnp.where(segq_ref[...] == segt_ref[...], d2, jnp.inf)
    colf = lax.broadcasted_iota(jnp.int32, (1, NPAD), 1).astype(jnp.float32)
    sel = jnp.zeros((B, NPAD), jnp.float32)
    for _ in range(K):
        m = jnp.min(d2m, axis=1, keepdims=True)
        idxf = jnp.min(jnp.where(d2m == m, colf, 1e9), axis=1, keepdims=True)
        chosen = colf == idxf
        sel = jnp.where(chosen & (m <= R2_NMS), sel + 1.0, sel)
        d2m = jnp.where(chosen, jnp.inf, d2m)
    cnt = jnp.maximum(jnp.sum(sel, axis=1, keepdims=True), 1.0)
    out_ref[...] = jnp.dot(sel, sv_ref[...], precision=HI,
                           preferred_element_type=jnp.float32) / cnt


def _rs_body(q_ref, st_ref, segq_ref, segt_ref, out_ref, *, r2, nvalid):
    q = q_ref[...]
    st = st_ref[...]
    a2 = _a2col(q)
    b2 = _b2row(st)
    d2 = jnp.maximum(
        a2 + b2 - 2.0 * jnp.dot(q, st, preferred_element_type=jnp.float32), 0.0)
    d2m = jnp.where(segq_ref[...] == segt_ref[...], d2, jnp.inf)
    colf = lax.broadcasted_iota(jnp.int32, (1, NPAD), 1).astype(jnp.float32)
    tcol = lax.broadcasted_iota(jnp.int32, (1, K), 1)
    acc = jnp.zeros((B, K), jnp.float32)
    for t in range(K):
        m = jnp.min(d2m, axis=1, keepdims=True)
        idxf = jnp.min(jnp.where(d2m == m, colf, 1e9), axis=1, keepdims=True)
        val = jnp.where(m <= r2, idxf, float(nvalid))
        acc = acc + jnp.where(tcol == t, val, 0.0)
        d2m = jnp.where(colf == idxf, jnp.inf, d2m)
    out_ref[...] = acc.astype(jnp.int32)


def _gn(x, gm, ge):
    mu = jnp.dot(x, gm, precision=HI, preferred_element_type=jnp.float32)
    d = x - jnp.dot(mu, ge, precision=HI, preferred_element_type=jnp.float32)
    var = jnp.dot(d * d, gm, precision=HI, preferred_element_type=jnp.float32)
    return d * lax.rsqrt(
        jnp.dot(var, ge, precision=HI, preferred_element_type=jnp.float32) + 1e-5)


def _res_body(idx_ref, x_ref, wa_ref, wb_ref, ws_ref, gm_ref, ge_ref,
              *rest, nvalid, with_keep):
    if with_keep:
        keep_ref, out_ref = rest
    else:
        (out_ref,) = rest
    ii = idx_ref[...]
    colv = lax.broadcasted_iota(jnp.int32, (1, NPAD), 1)
    sel = jnp.zeros((B, NPAD), jnp.float32)
    cntv = jnp.zeros((B, 1), jnp.float32)
    for t in range(K):
        it = lax.slice_in_dim(ii, t, t + 1, axis=1)
        sel = sel + jnp.where(colv == it, 1.0, 0.0)
        cntv = cntv + jnp.where(it < nvalid, 1.0, 0.0)
    cnt = jnp.maximum(cntv, 1.0)
    agg = jnp.dot(sel, x_ref[...], precision=HI,
                  preferred_element_type=jnp.float32) / cnt
    gm = gm_ref[...]
    ge = ge_ref[...]
    h0 = jnp.dot(agg, wa_ref[...], preferred_element_type=jnp.float32)
    h = jnp.maximum(_gn(h0, gm, ge), 0.0)
    h2 = (jnp.dot(h, wb_ref[...], preferred_element_type=jnp.float32)
          + jnp.dot(agg, ws_ref[...], preferred_element_type=jnp.float32))
    out = jnp.maximum(_gn(h2, gm, ge), 0.0)
    if with_keep:
        out = out * keep_ref[...]
    out_ref[...] = out


def _full(shape):
    return pl.BlockSpec(shape, lambda i: (0,) * len(shape))


def _rows(shape):
    return pl.BlockSpec(shape, lambda i: (i,) + (0,) * (len(shape) - 1))


def _cols(shape):
    return pl.BlockSpec(shape, lambda i: (0,) * (len(shape) - 1) + (i,))


def _vote_call(fpad, ppad, w1, b1, w2, b2):
    return pl.pallas_call(
        _vote_body,
        grid=(NB,),
        in_specs=[_rows((B, 256)), _rows((B, 128)), _full((256, 128)),
                  _full((1, 128)), _full((128, 128)), _full((1, 128))],
        out_specs=_rows((B, 128)),
        out_shape=jax.ShapeDtypeStruct((NPAD, 128), jnp.float32),
    )(fpad, ppad, w1, b1, w2, b2)


def _nms_call(sps, spst, segc, segr):
    return pl.pallas_call(
        _nms_body,
        grid=(NB,),
        in_specs=[_rows((B, 8)), _full((8, NPAD)), _rows((B, 1)),
                  _full((1, NPAD)), _cols((8, B)), _cols((1, B))],
        out_specs=[_rows((B, 1)), _full((8, 128))],
        out_shape=[jax.ShapeDtypeStruct((NPAD, 1), jnp.float32),
                   jax.ShapeDtypeStruct((8, 128), jnp.float32)],
        scratch_shapes=[pltpu.VMEM((NPAD, 1), jnp.float32),
                        pltpu.SMEM((2,), jnp.float32)],
    )(sps, spst, segc, segr, spst, segr)


def _knn_call(spo, spot, segc, segr):
    return pl.pallas_call(
        _knn_body,
        grid=(NB,),
        in_specs=[_rows((B, 8)), _full((8, NPAD)), _rows((B, 1)),
                  _full((1, NPAD)), _full((NPAD, 8))],
        out_specs=_rows((B, 8)),
        out_shape=jax.ShapeDtypeStruct((NPAD, 8), jnp.float32),
    )(spo, spot, segc, segr, spo)


def _rs_call(q, st, segc, segr, r2, nvalid):
    body = functools.partial(_rs_body, r2=r2, nvalid=nvalid)
    return pl.pallas_call(
        body,
        grid=(NB,),
        in_specs=[_rows((B, 8)), _full((8, NPAD)), _rows((B, 1)),
                  _full((1, NPAD))],
        out_specs=_rows((B, K)),
        out_shape=jax.ShapeDtypeStruct((NPAD, K), jnp.int32),
    )(q, st, segc, segr)


def _res_call(idx, x, wa, wb, ws, gm, ge, nvalid, keep=None):
    din = x.shape[1]
    dh = wa.shape[1]
    body = functools.partial(_res_body, nvalid=nvalid, with_keep=keep is not None)
    in_specs = [_rows((B, K)), _full((NPAD, din)), _full((din, dh)),
                _full((dh, dh)), _full((din, dh)), _full((dh, GROUPS)),
                _full((GROUPS, dh))]
    args = [idx, x, wa, wb, ws, gm, ge]
    if keep is not None:
        in_specs.append(_rows((B, 1)))
        args.append(keep)
    return pl.pallas_call(
        body,
        grid=(NB,),
        in_specs=in_specs,
        out_specs=_rows((B, dh)),
        out_shape=jax.ShapeDtypeStruct((NPAD, dh), jnp.float32),
    )(*args)


def _gmats(dh):
    g = jnp.arange(dh) // (dh // GROUPS)
    ge = (jnp.arange(GROUPS)[:, None] == g[None, :]).astype(jnp.float32)
    gm = ge.T / float(dh // GROUPS)
    return gm, ge


def kernel(feats, points, Wv1, bv1, Wv2, bv2, W1a, W1b, W1s, W2a, W2b, W2s,
           W3a, W3b, W3s, pos_length, anc_length):
    n, dim = feats.shape
    f32 = jnp.float32

    fpad = jnp.zeros((NPAD, dim), f32).at[:n].set(feats)
    ppad = jnp.zeros((NPAD, 128), f32).at[:n, :3].set(points)
    w2p = jnp.zeros((128, 128), f32).at[:, :3].set(Wv2)
    b1r = bv1.reshape(1, 128)
    b2r = jnp.zeros((1, 128), f32).at[0, :3].set(bv2)

    aug = _vote_call(fpad, ppad, Wv1, b1r, w2p, b2r)
    shifted = aug[:n, :3]
    scores = aug[:n, 3]

    seg = (jnp.arange(n) >= pos_length).astype(f32)
    segp = jnp.concatenate([seg, jnp.full((NPAD - n,), 2.0, f32)])
    segc = segp.reshape(NPAD, 1)
    segr = segp.reshape(1, NPAD)

    order = jnp.argsort(-scores)
    sps = jnp.zeros((NPAD, 8), f32).at[:n, :3].set(shifted[order])
    seg_s = jnp.concatenate([seg[order], jnp.full((NPAD - n,), 2.0, f32)])
    keep_s, len_out = _nms_call(sps, sps.T, seg_s.reshape(NPAD, 1),
                                seg_s.reshape(1, NPAD))
    keep_f = jnp.zeros((n,), f32).at[order].set(keep_s[:n, 0])
    keep = keep_f > 0.5
    length = jnp.round(len_out[0, :2]).astype(jnp.int32)

    spo = jnp.zeros((NPAD, 8), f32).at[:n, :3].set(shifted)
    caug = _knn_call(spo, spo.T, segc, segr)
    centers = caug[:n, :3]

    pto = jnp.zeros((NPAD, 8), f32).at[:n, :3].set(points)
    sub_idx = _rs_call(caug, pto.T, segc, segr, R2_SUB, n)
    nb_idx = _rs_call(caug, caug.T, segc, segr, R2_NB, n)

    gm256, ge256 = _gmats(256)
    gm512, ge512 = _gmats(512)
    keep_col = jnp.zeros((NPAD, 1), f32).at[:n, 0].set(keep_f)
    f1 = _res_call(sub_idx, fpad, W1a, W1b, W1s, gm256, ge256, n)
    f2 = _res_call(nb_idx, f1, W2a, W2b, W2s, gm512, ge512, n)
    f3 = _res_call(nb_idx, f2, W3a, W3b, W3s, gm512, ge512, n, keep=keep_col)

    return shifted, centers, length, keep, f3[:n]


# speedup vs baseline: 17.7377x; 1.2792x over previous
import functools

import jax
import jax.numpy as jnp
from jax import lax
from jax.experimental import pallas as pl
from jax.experimental.pallas import tpu as pltpu
from jax.experimental.pallas import tpu_sc as plsc

NPAD = 5120
B = 256
HI = lax.Precision.HIGHEST
NB = NPAD // B
K = 16
GROUPS = 32
R2_NMS = 0.1 * 0.1
R2_SUB = (0.0625 * 8.0) ** 2
R2_NB = (0.0625 * 16.0) ** 2


def _a2col(q):
    xq, yq, zq = q[:, 0:1], q[:, 1:2], q[:, 2:3]
    return (xq * xq + zq * zq) + yq * yq


def _b2row(st):
    xs, ys, zs = st[0:1, :], st[1:2, :], st[2:3, :]
    return (xs * xs + zs * zs) + ys * ys


def _vote_body(f_ref, p_ref, w1_ref, b1_ref, w2_ref, b2_ref, out_ref):
    x = f_ref[...]
    h = jnp.maximum(
        jnp.dot(x, w1_ref[...], preferred_element_type=jnp.float32) + b1_ref[...], 0.0)
    off = jnp.dot(h, w2_ref[...], preferred_element_type=jnp.float32) + b2_ref[...]
    score = jnp.sqrt(jnp.sum(x * x, axis=1, keepdims=True))
    col = lax.broadcasted_iota(jnp.int32, (1, 128), 1)
    out_ref[...] = off + p_ref[...] + jnp.where(col == 3, score, 0.0)


def _nms_body(sp_ref, spt_ref, seg_ref, segt_ref, sptb_ref, segtb_ref,
              keep_ref, len_ref, kv_ref, cnt_ref):
    b = pl.program_id(0)
    nb = pl.num_programs(0)

    @pl.when(b == 0)
    def _init():
        kv_ref[...] = jnp.zeros_like(kv_ref)
        cnt_ref[0] = 0.0
        cnt_ref[1] = 0.0

    q = sp_ref[...]
    st = spt_ref[...]
    a2 = _a2col(q)
    b2 = _b2row(st)
    d2 = jnp.maximum(
        a2 + b2 - 2.0 * jnp.dot(q, st, preferred_element_type=jnp.float32), 0.0)
    segq = seg_ref[...]
    adj = jnp.where((d2 < R2_NMS) & (segq == segt_ref[...]), 1.0, 0.0)
    sprior = jnp.dot(adj, kv_ref[...], preferred_element_type=jnp.float32)
    notprior = jnp.where(sprior > 0.0, 0.0, 1.0)

    stb = sptb_ref[...]
    b2in = _b2row(stb)
    d2in = jnp.maximum(
        a2 + b2in - 2.0 * jnp.dot(q, stb, preferred_element_type=jnp.float32), 0.0)
    rr = lax.broadcasted_iota(jnp.int32, (B, B), 0)
    cc = lax.broadcasted_iota(jnp.int32, (B, B), 1)
    adj_low = jnp.where(
        (d2in < R2_NMS) & (segq == segtb_ref[...]) & (cc < rr), 1.0, 0.0)

    def _cond(c):
        return c[1]

    def _body(c):
        kb, _ = c
        s = jnp.dot(adj_low, kb, preferred_element_type=jnp.float32)
        knew = notprior * jnp.where(s > 0.0, 0.0, 1.0)
        return knew, jnp.sum(jnp.abs(knew - kb)) > 0.0

    kb, _ = lax.while_loop(_cond, _body, (notprior, jnp.asarray(True)))

    kv_ref[pl.ds(b * B, B), :] = kb
    keep_ref[...] = kb
    cnt_ref[0] = cnt_ref[0] + jnp.sum(kb * jnp.where(segq == 0.0, 1.0, 0.0))
    cnt_ref[1] = cnt_ref[1] + jnp.sum(kb * jnp.where(segq == 1.0, 1.0, 0.0))

    @pl.when(b == nb - 1)
    def _fin():
        colv = lax.broadcasted_iota(jnp.int32, (8, 128), 1)
        len_ref[...] = jnp.where(
            colv == 0, cnt_ref[0], jnp.where(colv == 1, cnt_ref[1], 0.0))


def _knn_body(q_ref, st_ref, segq_ref, segt_ref, sv_ref, out_ref):
    q = q_ref[...]
    st = st_ref[...]
    a2 = _a2col(q)
    b2 = _b2row(st)
    d2 = jnp.maximum(
        a2 + b2 - 2.0 * jnp.dot(q, st, preferred_element_type=jnp.float32), 0.0)
    d2m = jnp.where(segq_ref[...] == segt_ref[...], d2, jnp.inf)
    colf = lax.broadcasted_iota(jnp.int32, (1, NPAD), 1).astype(jnp.float32)
    sel = jnp.zeros((B, NPAD), jnp.float32)
    for _ in range(K):
        m = jnp.min(d2m, axis=1, keepdims=True)
        idxf = jnp.min(jnp.where(d2m == m, colf, 1e9), axis=1, keepdims=True)
        chosen = colf == idxf
        sel = jnp.where(chosen & (m <= R2_NMS), sel + 1.0, sel)
        d2m = jnp.where(chosen, jnp.inf, d2m)
    cnt = jnp.maximum(jnp.sum(sel, axis=1, keepdims=True), 1.0)
    out_ref[...] = jnp.dot(sel, sv_ref[...], precision=HI,
                           preferred_element_type=jnp.float32) / cnt


def _rs_body(q_ref, st_ref, segq_ref, segt_ref, out_ref, *, r2, nvalid):
    q = q_ref[...]
    st = st_ref[...]
    a2 = _a2col(q)
    b2 = _b2row(st)
    d2 = jnp.maximum(
        a2 + b2 - 2.0 * jnp.dot(q, st, preferred_element_type=jnp.float32), 0.0)
    d2m = jnp.where(segq_ref[...] == segt_ref[...], d2, jnp.inf)
    colf = lax.broadcasted_iota(jnp.int32, (1, NPAD), 1).astype(jnp.float32)
    tcol = lax.broadcasted_iota(jnp.int32, (1, K), 1)
    acc = jnp.zeros((B, K), jnp.float32)
    for t in range(K):
        m = jnp.min(d2m, axis=1, keepdims=True)
        idxf = jnp.min(jnp.where(d2m == m, colf, 1e9), axis=1, keepdims=True)
        val = jnp.where(m <= r2, idxf, float(nvalid))
        acc = acc + jnp.where(tcol == t, val, 0.0)
        d2m = jnp.where(colf == idxf, jnp.inf, d2m)
    out_ref[...] = acc.astype(jnp.int32)


def _sc_agg_call(idx, table, d):
    info = plsc.get_sparse_core_info()
    nw = info.num_cores * info.num_subcores
    qpw = NPAD // nw
    ch = 8 if d <= 256 else 4
    nch = qpw // ch
    nlane = d // 16
    mesh = plsc.VectorSubcoreMesh(core_axis_name="c", subcore_axis_name="s")

    @functools.partial(
        pl.kernel, mesh=mesh,
        out_type=jax.ShapeDtypeStruct((NPAD, d), jnp.float32),
        scratch_types=[
            pltpu.VMEM((qpw, K), jnp.int32),
            pltpu.VMEM((ch * K, d), jnp.float32),
            pltpu.VMEM((ch, d), jnp.float32),
            pltpu.SemaphoreType.DMA,
        ],
    )
    def k(idx_hbm, table_hbm, out_hbm, idx_v, rows_v, acc_v, sem):
        wid = lax.axis_index("s") * info.num_cores + lax.axis_index("c")
        base = wid * qpw
        pltpu.sync_copy(idx_hbm.at[pl.ds(base, qpw), :], idx_v)

        def chunk_body(ci, carry):
            q0 = ci * ch
            cps = []
            for q in range(ch):
                cps.append(pltpu.async_copy(
                    table_hbm.at[idx_v.at[q0 + q]],
                    rows_v.at[pl.ds(q * K, K), :], sem))
            for cp in cps:
                cp.wait()
            for q in range(ch):
                def nb_body(j, acc):
                    return [acc[c] + rows_v[q * K + j, pl.ds(c * 16, 16)]
                            for c in range(nlane)]
                acc0 = [rows_v[q * K, pl.ds(c * 16, 16)] for c in range(nlane)]
                acc = lax.fori_loop(1, K, nb_body, acc0)
                for c in range(nlane):
                    acc_v[q, pl.ds(c * 16, 16)] = acc[c]
            pltpu.sync_copy(acc_v, out_hbm.at[pl.ds(base + q0, ch), :])
            return carry

        lax.fori_loop(0, nch, chunk_body, 0)

    return k(idx, table)


def _gn(x, gm, ge):
    mu = jnp.dot(x, gm, precision=HI, preferred_element_type=jnp.float32)
    d = x - jnp.dot(mu, ge, precision=HI, preferred_element_type=jnp.float32)
    var = jnp.dot(d * d, gm, precision=HI, preferred_element_type=jnp.float32)
    return d * lax.rsqrt(
        jnp.dot(var, ge, precision=HI, preferred_element_type=jnp.float32) + 1e-5)


def _res_body_sc(idx_ref, asum_ref, wa_ref, wb_ref, ws_ref, gm_ref, ge_ref,
                 *rest, nvalid, with_keep):
    if with_keep:
        keep_ref, out_ref = rest
    else:
        (out_ref,) = rest
    ii = idx_ref[...]
    cntv = jnp.zeros((B, 1), jnp.float32)
    for t in range(K):
        it = lax.slice_in_dim(ii, t, t + 1, axis=1)
        cntv = cntv + jnp.where(it < nvalid, 1.0, 0.0)
    agg = asum_ref[...] / jnp.maximum(cntv, 1.0)
    gm = gm_ref[...]
    ge = ge_ref[...]
    h0 = jnp.dot(agg, wa_ref[...], preferred_element_type=jnp.float32)
    h = jnp.maximum(_gn(h0, gm, ge), 0.0)
    h2 = (jnp.dot(h, wb_ref[...], preferred_element_type=jnp.float32)
          + jnp.dot(agg, ws_ref[...], preferred_element_type=jnp.float32))
    out = jnp.maximum(_gn(h2, gm, ge), 0.0)
    if with_keep:
        out = out * keep_ref[...]
    out_ref[...] = out


def _res_sc_call(idx, asum, wa, wb, ws, gm, ge, nvalid, keep=None):
    din = asum.shape[1]
    dh = wa.shape[1]
    body = functools.partial(_res_body_sc, nvalid=nvalid,
                             with_keep=keep is not None)
    in_specs = [_rows((B, K)), _rows((B, din)), _full((din, dh)),
                _full((dh, dh)), _full((din, dh)), _full((dh, GROUPS)),
                _full((GROUPS, dh))]
    args = [idx, asum, wa, wb, ws, gm, ge]
    if keep is not None:
        in_specs.append(_rows((B, 1)))
        args.append(keep)
    return pl.pallas_call(
        body,
        grid=(NB,),
        in_specs=in_specs,
        out_specs=_rows((B, dh)),
        out_shape=jax.ShapeDtypeStruct((NPAD, dh), jnp.float32),
    )(*args)


def _res_body(idx_ref, x_ref, wa_ref, wb_ref, ws_ref, gm_ref, ge_ref,
              *rest, nvalid, with_keep):
    if with_keep:
        keep_ref, out_ref = rest
    else:
        (out_ref,) = rest
    ii = idx_ref[...]
    colv = lax.broadcasted_iota(jnp.int32, (1, NPAD), 1)
    sel = jnp.zeros((B, NPAD), jnp.float32)
    cntv = jnp.zeros((B, 1), jnp.float32)
    for t in range(K):
        it = lax.slice_in_dim(ii, t, t + 1, axis=1)
        sel = sel + jnp.where(colv == it, 1.0, 0.0)
        cntv = cntv + jnp.where(it < nvalid, 1.0, 0.0)
    cnt = jnp.maximum(cntv, 1.0)
    agg = jnp.dot(sel, x_ref[...], precision=HI,
                  preferred_element_type=jnp.float32) / cnt
    gm = gm_ref[...]
    ge = ge_ref[...]
    h0 = jnp.dot(agg, wa_ref[...], preferred_element_type=jnp.float32)
    h = jnp.maximum(_gn(h0, gm, ge), 0.0)
    h2 = (jnp.dot(h, wb_ref[...], preferred_element_type=jnp.float32)
          + jnp.dot(agg, ws_ref[...], preferred_element_type=jnp.float32))
    out = jnp.maximum(_gn(h2, gm, ge), 0.0)
    if with_keep:
        out = out * keep_ref[...]
    out_ref[...] = out


def _full(shape):
    return pl.BlockSpec(shape, lambda i: (0,) * len(shape))


def _rows(shape):
    return pl.BlockSpec(shape, lambda i: (i,) + (0,) * (len(shape) - 1))


def _cols(shape):
    return pl.BlockSpec(shape, lambda i: (0,) * (len(shape) - 1) + (i,))


def _vote_call(fpad, ppad, w1, b1, w2, b2):
    return pl.pallas_call(
        _vote_body,
        grid=(NB,),
        in_specs=[_rows((B, 256)), _rows((B, 128)), _full((256, 128)),
                  _full((1, 128)), _full((128, 128)), _full((1, 128))],
        out_specs=_rows((B, 128)),
        out_shape=jax.ShapeDtypeStruct((NPAD, 128), jnp.float32),
    )(fpad, ppad, w1, b1, w2, b2)


def _nms_call(sps, spst, segc, segr):
    return pl.pallas_call(
        _nms_body,
        grid=(NB,),
        in_specs=[_rows((B, 8)), _full((8, NPAD)), _rows((B, 1)),
                  _full((1, NPAD)), _cols((8, B)), _cols((1, B))],
        out_specs=[_rows((B, 1)), _full((8, 128))],
        out_shape=[jax.ShapeDtypeStruct((NPAD, 1), jnp.float32),
                   jax.ShapeDtypeStruct((8, 128), jnp.float32)],
        scratch_shapes=[pltpu.VMEM((NPAD, 1), jnp.float32),
                        pltpu.SMEM((2,), jnp.float32)],
    )(sps, spst, segc, segr, spst, segr)


def _knn_call(spo, spot, segc, segr):
    return pl.pallas_call(
        _knn_body,
        grid=(NB,),
        in_specs=[_rows((B, 8)), _full((8, NPAD)), _rows((B, 1)),
                  _full((1, NPAD)), _full((NPAD, 8))],
        out_specs=_rows((B, 8)),
        out_shape=jax.ShapeDtypeStruct((NPAD, 8), jnp.float32),
    )(spo, spot, segc, segr, spo)


def _rs_call(q, st, segc, segr, r2, nvalid):
    body = functools.partial(_rs_body, r2=r2, nvalid=nvalid)
    return pl.pallas_call(
        body,
        grid=(NB,),
        in_specs=[_rows((B, 8)), _full((8, NPAD)), _rows((B, 1)),
                  _full((1, NPAD))],
        out_specs=_rows((B, K)),
        out_shape=jax.ShapeDtypeStruct((NPAD, K), jnp.int32),
    )(q, st, segc, segr)


def _res_call(idx, x, wa, wb, ws, gm, ge, nvalid, keep=None):
    din = x.shape[1]
    dh = wa.shape[1]
    body = functools.partial(_res_body, nvalid=nvalid, with_keep=keep is not None)
    in_specs = [_rows((B, K)), _full((NPAD, din)), _full((din, dh)),
                _full((dh, dh)), _full((din, dh)), _full((dh, GROUPS)),
                _full((GROUPS, dh))]
    args = [idx, x, wa, wb, ws, gm, ge]
    if keep is not None:
        in_specs.append(_rows((B, 1)))
        args.append(keep)
    return pl.pallas_call(
        body,
        grid=(NB,),
        in_specs=in_specs,
        out_specs=_rows((B, dh)),
        out_shape=jax.ShapeDtypeStruct((NPAD, dh), jnp.float32),
    )(*args)


def _gmats(dh):
    g = jnp.arange(dh) // (dh // GROUPS)
    ge = (jnp.arange(GROUPS)[:, None] == g[None, :]).astype(jnp.float32)
    gm = ge.T / float(dh // GROUPS)
    return gm, ge


def kernel(feats, points, Wv1, bv1, Wv2, bv2, W1a, W1b, W1s, W2a, W2b, W2s,
           W3a, W3b, W3s, pos_length, anc_length):
    n, dim = feats.shape
    f32 = jnp.float32

    fpad = jnp.zeros((NPAD, dim), f32).at[:n].set(feats)
    ppad = jnp.zeros((NPAD, 128), f32).at[:n, :3].set(points)
    w2p = jnp.zeros((128, 128), f32).at[:, :3].set(Wv2)
    b1r = bv1.reshape(1, 128)
    b2r = jnp.zeros((1, 128), f32).at[0, :3].set(bv2)

    aug = _vote_call(fpad, ppad, Wv1, b1r, w2p, b2r)
    shifted = aug[:n, :3]
    scores = aug[:n, 3]

    seg = (jnp.arange(n) >= pos_length).astype(f32)
    segp = jnp.concatenate([seg, jnp.full((NPAD - n,), 2.0, f32)])
    segc = segp.reshape(NPAD, 1)
    segr = segp.reshape(1, NPAD)

    order = jnp.argsort(-scores)
    sps = jnp.zeros((NPAD, 8), f32).at[:n, :3].set(shifted[order])
    seg_s = jnp.concatenate([seg[order], jnp.full((NPAD - n,), 2.0, f32)])
    keep_s, len_out = _nms_call(sps, sps.T, seg_s.reshape(NPAD, 1),
                                seg_s.reshape(1, NPAD))
    keep_f = jnp.zeros((n,), f32).at[order].set(keep_s[:n, 0])
    keep = keep_f > 0.5
    length = jnp.round(len_out[0, :2]).astype(jnp.int32)

    spo = jnp.zeros((NPAD, 8), f32).at[:n, :3].set(shifted)
    caug = _knn_call(spo, spo.T, segc, segr)
    centers = caug[:n, :3]

    pto = jnp.zeros((NPAD, 8), f32).at[:n, :3].set(points)
    sub_idx = _rs_call(caug, pto.T, segc, segr, R2_SUB, n)
    nb_idx = _rs_call(caug, caug.T, segc, segr, R2_NB, n)

    gm256, ge256 = _gmats(256)
    gm512, ge512 = _gmats(512)
    keep_col = jnp.zeros((NPAD, 1), f32).at[:n, 0].set(keep_f)
    a1 = _sc_agg_call(sub_idx, fpad, 256)
    f1 = _res_sc_call(sub_idx, a1, W1a, W1b, W1s, gm256, ge256, n)
    a2 = _sc_agg_call(nb_idx, f1, 256)
    f2 = _res_sc_call(nb_idx, a2, W2a, W2b, W2s, gm512, ge512, n)
    a3 = _sc_agg_call(nb_idx, f2, 512)
    f3 = _res_sc_call(nb_idx, a3, W3a, W3b, W3s, gm512, ge512, n, keep=keep_col)

    return shifted, centers, length, keep, f3[:n]


# speedup vs baseline: 25.1375x; 1.4172x over previous
import functools

import jax
import jax.numpy as jnp
from jax import lax
from jax.experimental import pallas as pl
from jax.experimental.pallas import tpu as pltpu
from jax.experimental.pallas import tpu_sc as plsc

NPAD = 5120
B = 256
HI = lax.Precision.HIGHEST
NB = NPAD // B
K = 16
GROUPS = 32
R2_NMS = 0.1 * 0.1
R2_SUB = (0.0625 * 8.0) ** 2
R2_NB = (0.0625 * 16.0) ** 2


def _a2col(q):
    xq, yq, zq = q[:, 0:1], q[:, 1:2], q[:, 2:3]
    return (xq * xq + zq * zq) + yq * yq


def _b2row(st):
    xs, ys, zs = st[0:1, :], st[1:2, :], st[2:3, :]
    return (xs * xs + zs * zs) + ys * ys


def _vote_body(f_ref, p_ref, w1_ref, b1_ref, w2_ref, b2_ref, out_ref):
    x = f_ref[...]
    h = jnp.maximum(
        jnp.dot(x, w1_ref[...], preferred_element_type=jnp.float32) + b1_ref[...], 0.0)
    off = jnp.dot(h, w2_ref[...], preferred_element_type=jnp.float32) + b2_ref[...]
    score = jnp.sqrt(jnp.sum(x * x, axis=1, keepdims=True))
    col = lax.broadcasted_iota(jnp.int32, (1, 128), 1)
    out_ref[...] = off + p_ref[...] + jnp.where(col == 3, score, 0.0)


def _nms_body(sp_ref, spt_ref, seg_ref, segt_ref, sptb_ref, segtb_ref,
              keep_ref, len_ref, kv_ref, cnt_ref):
    b = pl.program_id(0)
    nb = pl.num_programs(0)

    @pl.when(b == 0)
    def _init():
        kv_ref[...] = jnp.zeros_like(kv_ref)
        cnt_ref[0] = 0.0
        cnt_ref[1] = 0.0

    q = sp_ref[...]
    st = spt_ref[...]
    a2 = _a2col(q)
    b2 = _b2row(st)
    d2 = jnp.maximum(
        a2 + b2 - 2.0 * jnp.dot(q, st, preferred_element_type=jnp.float32), 0.0)
    segq = seg_ref[...]
    adj = jnp.where((d2 < R2_NMS) & (segq == segt_ref[...]), 1.0, 0.0)
    sprior = jnp.dot(adj, kv_ref[...], preferred_element_type=jnp.float32)
    notprior = jnp.where(sprior > 0.0, 0.0, 1.0)

    stb = sptb_ref[...]
    b2in = _b2row(stb)
    d2in = jnp.maximum(
        a2 + b2in - 2.0 * jnp.dot(q, stb, preferred_element_type=jnp.float32), 0.0)
    rr = lax.broadcasted_iota(jnp.int32, (B, B), 0)
    cc = lax.broadcasted_iota(jnp.int32, (B, B), 1)
    adj_low = jnp.where(
        (d2in < R2_NMS) & (segq == segtb_ref[...]) & (cc < rr), 1.0, 0.0)

    def _cond(c):
        return c[1]

    def _body(c):
        kb, _ = c
        s = jnp.dot(adj_low, kb, preferred_element_type=jnp.float32)
        knew = notprior * jnp.where(s > 0.0, 0.0, 1.0)
        return knew, jnp.sum(jnp.abs(knew - kb)) > 0.0

    kb, _ = lax.while_loop(_cond, _body, (notprior, jnp.asarray(True)))

    kv_ref[pl.ds(b * B, B), :] = kb
    keep_ref[...] = kb
    cnt_ref[0] = cnt_ref[0] + jnp.sum(kb * jnp.where(segq == 0.0, 1.0, 0.0))
    cnt_ref[1] = cnt_ref[1] + jnp.sum(kb * jnp.where(segq == 1.0, 1.0, 0.0))

    @pl.when(b == nb - 1)
    def _fin():
        colv = lax.broadcasted_iota(jnp.int32, (8, 128), 1)
        len_ref[...] = jnp.where(
            colv == 0, cnt_ref[0], jnp.where(colv == 1, cnt_ref[1], 0.0))


def _knn_body(q_ref, st_ref, segq_ref, segt_ref, sv_ref, out_ref):
    q = q_ref[...]
    st = st_ref[...]
    a2 = _a2col(q)
    b2 = _b2row(st)
    d2 = jnp.maximum(
        a2 + b2 - 2.0 * jnp.dot(q, st, preferred_element_type=jnp.float32), 0.0)
    d2m = jnp.where(segq_ref[...] == segt_ref[...], d2, jnp.inf)
    colf = lax.broadcasted_iota(jnp.int32, (1, NPAD), 1).astype(jnp.float32)
    within = d2m <= R2_NMS
    rowcnt = jnp.sum(jnp.where(within, 1.0, 0.0), axis=1, keepdims=True)

    def _fast(_):
        return jnp.where(within, 1.0, 0.0)

    def _slow(_):
        sel = jnp.zeros((B, NPAD), jnp.float32)
        d2x = d2m
        for _ in range(K):
            m = jnp.min(d2x, axis=1, keepdims=True)
            idxf = jnp.min(jnp.where(d2x == m, colf, 1e9), axis=1, keepdims=True)
            chosen = colf == idxf
            sel = jnp.where(chosen & (m <= R2_NMS), sel + 1.0, sel)
            d2x = jnp.where(chosen, jnp.inf, d2x)
        return sel

    sel = lax.cond(jnp.all(rowcnt <= float(K)), _fast, _slow, 0)
    cnt = jnp.maximum(jnp.sum(sel, axis=1, keepdims=True), 1.0)
    out_ref[...] = jnp.dot(sel, sv_ref[...], precision=HI,
                           preferred_element_type=jnp.float32) / cnt


def _rs_body(q_ref, st_ref, segq_ref, segt_ref, out_ref, *, r2, nvalid):
    q = q_ref[...]
    st = st_ref[...]
    a2 = _a2col(q)
    b2 = _b2row(st)
    d2 = jnp.maximum(
        a2 + b2 - 2.0 * jnp.dot(q, st, preferred_element_type=jnp.float32), 0.0)
    d2m = jnp.where(segq_ref[...] == segt_ref[...], d2, jnp.inf)
    colf = lax.broadcasted_iota(jnp.int32, (1, NPAD), 1).astype(jnp.float32)
    tcol = lax.broadcasted_iota(jnp.int32, (1, K), 1)
    acc = jnp.zeros((B, K), jnp.float32)
    for t in range(K):
        m = jnp.min(d2m, axis=1, keepdims=True)
        idxf = jnp.min(jnp.where(d2m == m, colf, 1e9), axis=1, keepdims=True)
        val = jnp.where(m <= r2, idxf, float(nvalid))
        acc = acc + jnp.where(tcol == t, val, 0.0)
        d2m = jnp.where(colf == idxf, jnp.inf, d2m)
    out_ref[...] = acc.astype(jnp.int32)


def _sc_agg_call(idx, table, d):
    info = plsc.get_sparse_core_info()
    nw = info.num_cores * info.num_subcores
    qpw = NPAD // nw
    ch = 8 if d <= 256 else 4
    nch = qpw // ch
    nlane = d // 16
    mesh = plsc.VectorSubcoreMesh(core_axis_name="c", subcore_axis_name="s")

    @functools.partial(
        pl.kernel, mesh=mesh,
        out_type=jax.ShapeDtypeStruct((NPAD, d), jnp.float32),
        scratch_types=[
            pltpu.VMEM((qpw, K), jnp.int32),
            pltpu.VMEM((ch * K, d), jnp.float32),
            pltpu.VMEM((ch, d), jnp.float32),
            pltpu.SemaphoreType.DMA,
        ],
    )
    def k(idx_hbm, table_hbm, out_hbm, idx_v, rows_v, acc_v, sem):
        wid = lax.axis_index("s") * info.num_cores + lax.axis_index("c")
        base = wid * qpw
        pltpu.sync_copy(idx_hbm.at[pl.ds(base, qpw), :], idx_v)

        def chunk_body(ci, carry):
            q0 = ci * ch
            cps = []
            for q in range(ch):
                cps.append(pltpu.async_copy(
                    table_hbm.at[idx_v.at[q0 + q]],
                    rows_v.at[pl.ds(q * K, K), :], sem))
            for cp in cps:
                cp.wait()
            for q in range(ch):
                def nb_body(j, acc):
                    return [acc[c] + rows_v[q * K + j, pl.ds(c * 16, 16)]
                            for c in range(nlane)]
                acc0 = [rows_v[q * K, pl.ds(c * 16, 16)] for c in range(nlane)]
                acc = lax.fori_loop(1, K, nb_body, acc0)
                for c in range(nlane):
                    acc_v[q, pl.ds(c * 16, 16)] = acc[c]
            pltpu.sync_copy(acc_v, out_hbm.at[pl.ds(base + q0, ch), :])
            return carry

        lax.fori_loop(0, nch, chunk_body, 0)

    return k(idx, table)


def _gn(x, gm, ge):
    mu = jnp.dot(x, gm, precision=HI, preferred_element_type=jnp.float32)
    d = x - jnp.dot(mu, ge, precision=HI, preferred_element_type=jnp.float32)
    var = jnp.dot(d * d, gm, precision=HI, preferred_element_type=jnp.float32)
    return d * lax.rsqrt(
        jnp.dot(var, ge, precision=HI, preferred_element_type=jnp.float32) + 1e-5)


def _res_body_sc(idx_ref, asum_ref, wa_ref, wb_ref, ws_ref, gm_ref, ge_ref,
                 *rest, nvalid, with_keep):
    if with_keep:
        keep_ref, out_ref = rest
    else:
        (out_ref,) = rest
    ii = idx_ref[...]
    cntv = jnp.zeros((B, 1), jnp.float32)
    for t in range(K):
        it = lax.slice_in_dim(ii, t, t + 1, axis=1)
        cntv = cntv + jnp.where(it < nvalid, 1.0, 0.0)
    agg = asum_ref[...] / jnp.maximum(cntv, 1.0)
    gm = gm_ref[...]
    ge = ge_ref[...]
    h0 = jnp.dot(agg, wa_ref[...], preferred_element_type=jnp.float32)
    h = jnp.maximum(_gn(h0, gm, ge), 0.0)
    h2 = (jnp.dot(h, wb_ref[...], preferred_element_type=jnp.float32)
          + jnp.dot(agg, ws_ref[...], preferred_element_type=jnp.float32))
    out = jnp.maximum(_gn(h2, gm, ge), 0.0)
    if with_keep:
        out = out * keep_ref[...]
    out_ref[...] = out


def _res_sc_call(idx, asum, wa, wb, ws, gm, ge, nvalid, keep=None):
    din = asum.shape[1]
    dh = wa.shape[1]
    body = functools.partial(_res_body_sc, nvalid=nvalid,
                             with_keep=keep is not None)
    in_specs = [_rows((B, K)), _rows((B, din)), _full((din, dh)),
                _full((dh, dh)), _full((din, dh)), _full((dh, GROUPS)),
                _full((GROUPS, dh))]
    args = [idx, asum, wa, wb, ws, gm, ge]
    if keep is not None:
        in_specs.append(_rows((B, 1)))
        args.append(keep)
    return pl.pallas_call(
        body,
        grid=(NB,),
        in_specs=in_specs,
        out_specs=_rows((B, dh)),
        out_shape=jax.ShapeDtypeStruct((NPAD, dh), jnp.float32),
    )(*args)


def _res_body(idx_ref, x_ref, wa_ref, wb_ref, ws_ref, gm_ref, ge_ref,
              *rest, nvalid, with_keep):
    if with_keep:
        keep_ref, out_ref = rest
    else:
        (out_ref,) = rest
    ii = idx_ref[...]
    colv = lax.broadcasted_iota(jnp.int32, (1, NPAD), 1)
    sel = jnp.zeros((B, NPAD), jnp.float32)
    cntv = jnp.zeros((B, 1), jnp.float32)
    for t in range(K):
        it = lax.slice_in_dim(ii, t, t + 1, axis=1)
        sel = sel + jnp.where(colv == it, 1.0, 0.0)
        cntv = cntv + jnp.where(it < nvalid, 1.0, 0.0)
    cnt = jnp.maximum(cntv, 1.0)
    agg = jnp.dot(sel, x_ref[...], precision=HI,
                  preferred_element_type=jnp.float32) / cnt
    gm = gm_ref[...]
    ge = ge_ref[...]
    h0 = jnp.dot(agg, wa_ref[...], preferred_element_type=jnp.float32)
    h = jnp.maximum(_gn(h0, gm, ge), 0.0)
    h2 = (jnp.dot(h, wb_ref[...], preferred_element_type=jnp.float32)
          + jnp.dot(agg, ws_ref[...], preferred_element_type=jnp.float32))
    out = jnp.maximum(_gn(h2, gm, ge), 0.0)
    if with_keep:
        out = out * keep_ref[...]
    out_ref[...] = out


def _full(shape):
    return pl.BlockSpec(shape, lambda i: (0,) * len(shape))


def _rows(shape):
    return pl.BlockSpec(shape, lambda i: (i,) + (0,) * (len(shape) - 1))


def _cols(shape):
    return pl.BlockSpec(shape, lambda i: (0,) * (len(shape) - 1) + (i,))


def _vote_call(fpad, ppad, w1, b1, w2, b2):
    return pl.pallas_call(
        _vote_body,
        grid=(NB,),
        in_specs=[_rows((B, 256)), _rows((B, 128)), _full((256, 128)),
                  _full((1, 128)), _full((128, 128)), _full((1, 128))],
        out_specs=_rows((B, 128)),
        out_shape=jax.ShapeDtypeStruct((NPAD, 128), jnp.float32),
    )(fpad, ppad, w1, b1, w2, b2)


def _nms_call(sps, spst, segc, segr):
    return pl.pallas_call(
        _nms_body,
        grid=(NB,),
        in_specs=[_rows((B, 8)), _full((8, NPAD)), _rows((B, 1)),
                  _full((1, NPAD)), _cols((8, B)), _cols((1, B))],
        out_specs=[_rows((B, 1)), _full((8, 128))],
        out_shape=[jax.ShapeDtypeStruct((NPAD, 1), jnp.float32),
                   jax.ShapeDtypeStruct((8, 128), jnp.float32)],
        scratch_shapes=[pltpu.VMEM((NPAD, 1), jnp.float32),
                        pltpu.SMEM((2,), jnp.float32)],
    )(sps, spst, segc, segr, spst, segr)


def _knn_call(spo, spot, segc, segr):
    return pl.pallas_call(
        _knn_body,
        grid=(NB,),
        in_specs=[_rows((B, 8)), _full((8, NPAD)), _rows((B, 1)),
                  _full((1, NPAD)), _full((NPAD, 8))],
        out_specs=_rows((B, 8)),
        out_shape=jax.ShapeDtypeStruct((NPAD, 8), jnp.float32),
    )(spo, spot, segc, segr, spo)


def _rs_call(q, st, segc, segr, r2, nvalid):
    body = functools.partial(_rs_body, r2=r2, nvalid=nvalid)
    return pl.pallas_call(
        body,
        grid=(NB,),
        in_specs=[_rows((B, 8)), _full((8, NPAD)), _rows((B, 1)),
                  _full((1, NPAD))],
        out_specs=_rows((B, K)),
        out_shape=jax.ShapeDtypeStruct((NPAD, K), jnp.int32),
    )(q, st, segc, segr)


def _res_call(idx, x, wa, wb, ws, gm, ge, nvalid, keep=None):
    din = x.shape[1]
    dh = wa.shape[1]
    body = functools.partial(_res_body, nvalid=nvalid, with_keep=keep is not None)
    in_specs = [_rows((B, K)), _full((NPAD, din)), _full((din, dh)),
                _full((dh, dh)), _full((din, dh)), _full((dh, GROUPS)),
                _full((GROUPS, dh))]
    args = [idx, x, wa, wb, ws, gm, ge]
    if keep is not None:
        in_specs.append(_rows((B, 1)))
        args.append(keep)
    return pl.pallas_call(
        body,
        grid=(NB,),
        in_specs=in_specs,
        out_specs=_rows((B, dh)),
        out_shape=jax.ShapeDtypeStruct((NPAD, dh), jnp.float32),
    )(*args)


def _gmats(dh):
    g = jnp.arange(dh) // (dh // GROUPS)
    ge = (jnp.arange(GROUPS)[:, None] == g[None, :]).astype(jnp.float32)
    gm = ge.T / float(dh // GROUPS)
    return gm, ge


def kernel(feats, points, Wv1, bv1, Wv2, bv2, W1a, W1b, W1s, W2a, W2b, W2s,
           W3a, W3b, W3s, pos_length, anc_length):
    n, dim = feats.shape
    f32 = jnp.float32

    fpad = jnp.zeros((NPAD, dim), f32).at[:n].set(feats)
    ppad = jnp.zeros((NPAD, 128), f32).at[:n, :3].set(points)
    w2p = jnp.zeros((128, 128), f32).at[:, :3].set(Wv2)
    b1r = bv1.reshape(1, 128)
    b2r = jnp.zeros((1, 128), f32).at[0, :3].set(bv2)

    aug = _vote_call(fpad, ppad, Wv1, b1r, w2p, b2r)
    shifted = aug[:n, :3]
    scores = aug[:n, 3]

    seg = (jnp.arange(n) >= pos_length).astype(f32)
    segp = jnp.concatenate([seg, jnp.full((NPAD - n,), 2.0, f32)])
    segc = segp.reshape(NPAD, 1)
    segr = segp.reshape(1, NPAD)

    order = jnp.argsort(-scores)
    sps = jnp.zeros((NPAD, 8), f32).at[:n, :3].set(shifted[order])
    seg_s = jnp.concatenate([seg[order], jnp.full((NPAD - n,), 2.0, f32)])
    keep_s, len_out = _nms_call(sps, sps.T, seg_s.reshape(NPAD, 1),
                                seg_s.reshape(1, NPAD))
    keep_f = jnp.zeros((n,), f32).at[order].set(keep_s[:n, 0])
    keep = keep_f > 0.5
    length = jnp.round(len_out[0, :2]).astype(jnp.int32)

    spo = jnp.zeros((NPAD, 8), f32).at[:n, :3].set(shifted)
    caug = _knn_call(spo, spo.T, segc, segr)
    centers = caug[:n, :3]

    pto = jnp.zeros((NPAD, 8), f32).at[:n, :3].set(points)
    sub_idx = _rs_call(caug, pto.T, segc, segr, R2_SUB, n)
    nb_idx = _rs_call(caug, caug.T, segc, segr, R2_NB, n)

    gm256, ge256 = _gmats(256)
    gm512, ge512 = _gmats(512)
    keep_col = jnp.zeros((NPAD, 1), f32).at[:n, 0].set(keep_f)
    a1 = _sc_agg_call(sub_idx, fpad, 256)
    f1 = _res_sc_call(sub_idx, a1, W1a, W1b, W1s, gm256, ge256, n)
    a2 = _sc_agg_call(nb_idx, f1, 256)
    f2 = _res_sc_call(nb_idx, a2, W2a, W2b, W2s, gm512, ge512, n)
    a3 = _sc_agg_call(nb_idx, f2, 512)
    f3 = _res_sc_call(nb_idx, a3, W3a, W3b, W3s, gm512, ge512, n, keep=keep_col)

    return shifted, centers, length, keep, f3[:n]
